# baseline (speedup 1.0000x reference)
import numpy as np
import jax
import jax.numpy as jnp
from jax.experimental import pallas as pl
from jax.experimental.pallas import tpu as pltpu


def _conv1_tables(w1):
    j1 = np.zeros((2, 128, 32, 3), np.float32)
    for par in range(2):
        for u in range(32):
            for kw in range(3):
                jp = 2 * u + par + kw - 1
                if 0 <= jp <= 63:
                    j1[par, jp + 8, u, kw] = 1.0
    halves = [jnp.einsum('quk,hkcd->chqud', jnp.asarray(j1[p]),
                         w1).reshape(1152, 1024) for p in range(2)]
    return jnp.concatenate(halves, axis=1).astype(jnp.bfloat16)


def _conv2_tables(w2):
    j2 = np.zeros((2, 12, 4, 3), np.float32)
    for par in range(2):
        for v in range(4):
            for kw in range(3):
                q2 = 2 * v + par + kw
                if 0 <= q2 <= 9:
                    j2[par, q2, v, kw] = 1.0
    halves = [jnp.einsum('qvk,hkcd->hqcvd', jnp.asarray(j2[p]),
                         w2).reshape(1152, 256) for p in range(2)]
    return jnp.concatenate(halves, axis=1).astype(jnp.bfloat16)


def _row_select_1():
    s = np.zeros((32, 64), np.float32)
    for k in range(32):
        s[k, 2 * k] = 1.0
    return jnp.asarray(s, dtype=jnp.bfloat16)


def _row_select_2():
    s = np.zeros((64, 128), np.float32)
    for g in range(4):
        for h in range(16):
            s[g * 16 + h, g * 32 + 2 * h] = 1.0
    return jnp.asarray(s, dtype=jnp.bfloat16)


def _make_conv_kernel(G):
    def _conv_kernel(x_ref, t1_ref, b1_ref, t2_ref, b2_ref, s1_ref, s2_ref,
                     o_ref, p1_ref, a2_ref, p2_ref, y1_ref, y2_ref):
        f32 = jnp.float32
        bf16 = jnp.bfloat16

        zrow = jnp.zeros((1, 128), bf16)
        for i in range(G):
            r0 = i * 64
            for c in range(3):
                v = x_ref[i, c].astype(bf16)
                vp = jnp.concatenate([jnp.zeros((64, 8), bf16), v,
                                      jnp.zeros((64, 56), bf16)], axis=1)
                for kh in range(3):
                    t = c * 3 + kh
                    if kh == 0:
                        p1_ref[pl.ds(r0, 1), pl.ds(t * 128, 128)] = zrow
                        p1_ref[pl.ds(r0 + 1, 63), pl.ds(t * 128, 128)] = (
                            vp[0:63, :])
                    elif kh == 1:
                        p1_ref[pl.ds(r0, 64), pl.ds(t * 128, 128)] = vp
                    else:
                        p1_ref[pl.ds(r0, 63), pl.ds(t * 128, 128)] = vp[1:64, :]
                        p1_ref[pl.ds(r0 + 63, 1), pl.ds(t * 128, 128)] = zrow

        y1_ref[...] = jnp.dot(p1_ref[...], t1_ref[...],
                              preferred_element_type=f32).astype(bf16)

        for i in range(G):
            yv = y1_ref[pl.ds(i * 64, 64), :]
            m1 = jnp.maximum(yv[:, :1024], yv[:, 1024:])
            rm1 = jnp.maximum(m1[0:63, :], m1[1:64, :])
            rm1 = jnp.concatenate([rm1, jnp.zeros((1, 1024), bf16)], axis=0)
            a2 = jnp.dot(s1_ref[...], rm1, preferred_element_type=f32)
            a2 = jnp.maximum(a2 + b1_ref[...], 0.0)
            a2_ref[pl.ds(i * 32, 32), :] = jnp.concatenate(
                [jnp.zeros((32, 32), bf16), a2.astype(bf16),
                 jnp.zeros((32, 96), bf16)], axis=1)

        zrow2 = jnp.zeros((1, 384), bf16)
        for i in range(G):
            q0 = i * 128
            a0 = i * 32
            for g in range(4):
                for kh in range(3):
                    if kh == 0:
                        p2_ref[pl.ds(q0 + g * 32, 1), pl.ds(0, 384)] = zrow2
                        p2_ref[pl.ds(q0 + g * 32 + 1, 31), pl.ds(0, 384)] = (
                            a2_ref[pl.ds(a0, 31), pl.ds(256 * g, 384)])
                    elif kh == 1:
                        p2_ref[pl.ds(q0 + g * 32, 32), pl.ds(384, 384)] = (
                            a2_ref[pl.ds(a0, 32), pl.ds(256 * g, 384)])
                    else:
                        p2_ref[pl.ds(q0 + g * 32, 31), pl.ds(768, 384)] = (
                            a2_ref[pl.ds(a0 + 1, 31), pl.ds(256 * g, 384)])
                        p2_ref[pl.ds(q0 + g * 32 + 31, 1),
                               pl.ds(768, 384)] = zrow2

        y2_ref[...] = jnp.dot(p2_ref[...], t2_ref[...],
                              preferred_element_type=f32).astype(bf16)

        for i in range(G):
            yv = y2_ref[pl.ds(i * 128, 128), :]
            m2 = jnp.maximum(yv[:, :256], yv[:, 256:])
            rm2 = jnp.maximum(m2[0:127, :], m2[1:128, :])
            rm2 = jnp.concatenate([rm2, jnp.zeros((1, 256), bf16)], axis=0)
            hsel = jnp.dot(s2_ref[...], rm2, preferred_element_type=f32)
            out = jnp.maximum(hsel + b2_ref[...], 0.0).astype(o_ref.dtype)
            for g in range(4):
                o_ref[i, :, pl.ds(256 * g, 256)] = out[16 * g:16 * (g + 1), :]

    return _conv_kernel


def _conv_stage(x_nchw, t1, b1t, t2, b2t, s1, s2):
    B = x_nchw.shape[0]
    G = 16 if B % 16 == 0 else (8 if B % 8 == 0 else
                                (4 if B % 4 == 0 else (2 if B % 2 == 0 else 1)))
    return pl.pallas_call(
        _make_conv_kernel(G),
        out_shape=jax.ShapeDtypeStruct((B, 16, 1024), jnp.bfloat16),
        grid_spec=pltpu.PrefetchScalarGridSpec(
            num_scalar_prefetch=0,
            grid=(B // G,),
            in_specs=[
                pl.BlockSpec((G, 3, 64, 64), lambda b: (b, 0, 0, 0)),
                pl.BlockSpec((1152, 2048), lambda b: (0, 0)),
                pl.BlockSpec((1, 1024), lambda b: (0, 0)),
                pl.BlockSpec((1152, 512), lambda b: (0, 0)),
                pl.BlockSpec((1, 256), lambda b: (0, 0)),
                pl.BlockSpec((32, 64), lambda b: (0, 0)),
                pl.BlockSpec((64, 128), lambda b: (0, 0)),
            ],
            out_specs=pl.BlockSpec((G, 16, 1024), lambda b: (b, 0, 0)),
            scratch_shapes=[
                pltpu.VMEM((G * 64, 1152), jnp.bfloat16),
                pltpu.VMEM((G * 32, 1152), jnp.bfloat16),
                pltpu.VMEM((G * 128, 1152), jnp.bfloat16),
                pltpu.VMEM((G * 64, 2048), jnp.bfloat16),
                pltpu.VMEM((G * 128, 512), jnp.bfloat16),
            ],
        ),
        compiler_params=pltpu.CompilerParams(
            dimension_semantics=("parallel",),
        ),
    )(x_nchw, t1, b1t, t2, b2t, s1, s2)


def _mlp_kernel(x_ref, w1_ref, b1_ref, w2_ref, b2_ref, o_ref, acc_ref):
    k = pl.program_id(1)

    @pl.when(k == 0)
    def _():
        acc_ref[...] = jnp.zeros(acc_ref.shape, acc_ref.dtype)

    acc_ref[...] += jnp.dot(x_ref[...], w1_ref[...],
                            preferred_element_type=jnp.float32)

    @pl.when(k == pl.num_programs(1) - 1)
    def _():
        h = jnp.maximum(acc_ref[...] + b1_ref[...], 0.0)
        logits = jnp.dot(h.astype(jnp.bfloat16), w2_ref[...],
                         preferred_element_type=jnp.float32) + b2_ref[...]
        o_ref[...] = logits.astype(o_ref.dtype)


def _mlp_head(x, w1, b1, w2, b2, *, k_chunk=2048, b_block=256):
    B, K = x.shape
    b_block = min(b_block, B)
    n_hidden = w1.shape[1]
    n_out = w2.shape[1]
    return pl.pallas_call(
        _mlp_kernel,
        out_shape=jax.ShapeDtypeStruct((B, n_out), jnp.float32),
        grid_spec=pltpu.PrefetchScalarGridSpec(
            num_scalar_prefetch=0,
            grid=(B // b_block, K // k_chunk),
            in_specs=[
                pl.BlockSpec((b_block, k_chunk), lambda i, k: (i, k)),
                pl.BlockSpec((k_chunk, n_hidden), lambda i, k: (k, 0)),
                pl.BlockSpec((1, n_hidden), lambda i, k: (0, 0)),
                pl.BlockSpec((n_hidden, n_out), lambda i, k: (0, 0)),
                pl.BlockSpec((1, n_out), lambda i, k: (0, 0)),
            ],
            out_specs=pl.BlockSpec((b_block, n_out), lambda i, k: (i, 0)),
            scratch_shapes=[pltpu.VMEM((b_block, n_hidden), jnp.float32)],
        ),
        compiler_params=pltpu.CompilerParams(
            dimension_semantics=("parallel", "arbitrary"),
        ),
    )(x, w1, b1, w2, b2)


def kernel(w1, b1, w2, b2, fc1_w, fc1_b, fc2_w, fc2_b, x_nchw):
    t1 = _conv1_tables(w1)
    t2 = _conv2_tables(w2)
    b1t = jnp.tile(b1, (1, 32))
    b2t = jnp.tile(b2, (1, 4))
    h = _conv_stage(x_nchw, t1, b1t, t2, b2t, _row_select_1(),
                    _row_select_2())
    flat = h.reshape(h.shape[0], 16 * 16 * 64)
    return _mlp_head(flat, fc1_w.astype(jnp.bfloat16), fc1_b,
                     fc2_w.astype(jnp.bfloat16), fc2_b)


# speedup vs baseline: 1.0017x; 1.0017x over previous
import numpy as np
import jax
import jax.numpy as jnp
from jax.experimental import pallas as pl
from jax.experimental.pallas import tpu as pltpu


def _conv1_tables(w1):
    j1 = np.zeros((2, 128, 32, 3), np.float32)
    for par in range(2):
        for u in range(32):
            for kw in range(3):
                jp = 2 * u + par + kw - 1
                if 0 <= jp <= 63:
                    j1[par, jp + 8, u, kw] = 1.0
    halves = [jnp.einsum('quk,hkcd->chqud', jnp.asarray(j1[p]),
                         w1).reshape(1152, 1024) for p in range(2)]
    return jnp.concatenate(halves, axis=1).astype(jnp.bfloat16)


def _conv2_tables(w2):
    j2 = np.zeros((2, 12, 4, 3), np.float32)
    for par in range(2):
        for v in range(4):
            for kw in range(3):
                q2 = 2 * v + par + kw
                if 0 <= q2 <= 9:
                    j2[par, q2, v, kw] = 1.0
    halves = [jnp.einsum('qvk,hkcd->hqcvd', jnp.asarray(j2[p]),
                         w2).reshape(1152, 256) for p in range(2)]
    return jnp.concatenate(halves, axis=1).astype(jnp.bfloat16)


def _row_select_1():
    s = np.zeros((32, 64), np.float32)
    for k in range(32):
        s[k, 2 * k] = 1.0
    return jnp.asarray(s)


def _row_select_2():
    s = np.zeros((64, 128), np.float32)
    for g in range(4):
        for h in range(16):
            s[g * 16 + h, g * 32 + 2 * h] = 1.0
    return jnp.asarray(s)


def _make_conv_kernel(G):
    def _conv_kernel(x_ref, t1_ref, b1_ref, t2_ref, b2_ref, s1_ref, s2_ref,
                     o_ref, p1_ref, a2_ref, p2_ref, y1_ref, y2_ref):
        f32 = jnp.float32
        bf16 = jnp.bfloat16

        zrow = jnp.zeros((1, 128), bf16)
        for i in range(G):
            r0 = i * 64
            for c in range(3):
                v = x_ref[i, c].astype(bf16)
                vp = jnp.concatenate([jnp.zeros((64, 8), bf16), v,
                                      jnp.zeros((64, 56), bf16)], axis=1)
                for kh in range(3):
                    t = c * 3 + kh
                    if kh == 0:
                        p1_ref[pl.ds(r0, 1), pl.ds(t * 128, 128)] = zrow
                        p1_ref[pl.ds(r0 + 1, 63), pl.ds(t * 128, 128)] = (
                            vp[0:63, :])
                    elif kh == 1:
                        p1_ref[pl.ds(r0, 64), pl.ds(t * 128, 128)] = vp
                    else:
                        p1_ref[pl.ds(r0, 63), pl.ds(t * 128, 128)] = vp[1:64, :]
                        p1_ref[pl.ds(r0 + 63, 1), pl.ds(t * 128, 128)] = zrow

        y1_ref[...] = jnp.dot(p1_ref[...], t1_ref[...],
                              preferred_element_type=f32)

        for i in range(G):
            yv = y1_ref[pl.ds(i * 64, 64), :]
            m1 = jnp.maximum(yv[:, :1024], yv[:, 1024:])
            rm1 = jnp.maximum(m1[0:63, :], m1[1:64, :])
            rm1 = jnp.concatenate([rm1, jnp.zeros((1, 1024), f32)], axis=0)
            a2 = jnp.dot(s1_ref[...], rm1, preferred_element_type=f32)
            a2 = jnp.maximum(a2 + b1_ref[...], 0.0)
            a2_ref[pl.ds(i * 32, 32), :] = jnp.concatenate(
                [jnp.zeros((32, 32), bf16), a2.astype(bf16),
                 jnp.zeros((32, 96), bf16)], axis=1)

        zrow2 = jnp.zeros((1, 384), bf16)
        for i in range(G):
            q0 = i * 128
            a0 = i * 32
            for g in range(4):
                for kh in range(3):
                    if kh == 0:
                        p2_ref[pl.ds(q0 + g * 32, 1), pl.ds(0, 384)] = zrow2
                        p2_ref[pl.ds(q0 + g * 32 + 1, 31), pl.ds(0, 384)] = (
                            a2_ref[pl.ds(a0, 31), pl.ds(256 * g, 384)])
                    elif kh == 1:
                        p2_ref[pl.ds(q0 + g * 32, 32), pl.ds(384, 384)] = (
                            a2_ref[pl.ds(a0, 32), pl.ds(256 * g, 384)])
                    else:
                        p2_ref[pl.ds(q0 + g * 32, 31), pl.ds(768, 384)] = (
                            a2_ref[pl.ds(a0 + 1, 31), pl.ds(256 * g, 384)])
                        p2_ref[pl.ds(q0 + g * 32 + 31, 1),
                               pl.ds(768, 384)] = zrow2

        y2_ref[...] = jnp.dot(p2_ref[...], t2_ref[...],
                              preferred_element_type=f32)

        for i in range(G):
            yv = y2_ref[pl.ds(i * 128, 128), :]
            m2 = jnp.maximum(yv[:, :256], yv[:, 256:])
            rm2 = jnp.maximum(m2[0:127, :], m2[1:128, :])
            rm2 = jnp.concatenate([rm2, jnp.zeros((1, 256), f32)], axis=0)
            hsel = jnp.dot(s2_ref[...], rm2, preferred_element_type=f32)
            out = jnp.maximum(hsel + b2_ref[...], 0.0).astype(o_ref.dtype)
            for g in range(4):
                o_ref[i, :, pl.ds(256 * g, 256)] = out[16 * g:16 * (g + 1), :]

    return _conv_kernel


def _conv_stage(x_nchw, t1, b1t, t2, b2t, s1, s2):
    B = x_nchw.shape[0]
    G = 8 if B % 8 == 0 else (4 if B % 4 == 0 else (2 if B % 2 == 0 else 1))
    return pl.pallas_call(
        _make_conv_kernel(G),
        out_shape=jax.ShapeDtypeStruct((B, 16, 1024), jnp.bfloat16),
        grid_spec=pltpu.PrefetchScalarGridSpec(
            num_scalar_prefetch=0,
            grid=(B // G,),
            in_specs=[
                pl.BlockSpec((G, 3, 64, 64), lambda b: (b, 0, 0, 0)),
                pl.BlockSpec((1152, 2048), lambda b: (0, 0)),
                pl.BlockSpec((1, 1024), lambda b: (0, 0)),
                pl.BlockSpec((1152, 512), lambda b: (0, 0)),
                pl.BlockSpec((1, 256), lambda b: (0, 0)),
                pl.BlockSpec((32, 64), lambda b: (0, 0)),
                pl.BlockSpec((64, 128), lambda b: (0, 0)),
            ],
            out_specs=pl.BlockSpec((G, 16, 1024), lambda b: (b, 0, 0)),
            scratch_shapes=[
                pltpu.VMEM((G * 64, 1152), jnp.bfloat16),
                pltpu.VMEM((G * 32, 1152), jnp.bfloat16),
                pltpu.VMEM((G * 128, 1152), jnp.bfloat16),
                pltpu.VMEM((G * 64, 2048), jnp.float32),
                pltpu.VMEM((G * 128, 512), jnp.float32),
            ],
        ),
        compiler_params=pltpu.CompilerParams(
            dimension_semantics=("parallel",),
        ),
    )(x_nchw, t1, b1t, t2, b2t, s1, s2)


def _mlp_kernel(x_ref, w1_ref, b1_ref, w2_ref, b2_ref, o_ref, acc_ref):
    k = pl.program_id(1)

    @pl.when(k == 0)
    def _():
        acc_ref[...] = jnp.zeros(acc_ref.shape, acc_ref.dtype)

    acc_ref[...] += jnp.dot(x_ref[...], w1_ref[...],
                            preferred_element_type=jnp.float32)

    @pl.when(k == pl.num_programs(1) - 1)
    def _():
        h = jnp.maximum(acc_ref[...] + b1_ref[...], 0.0)
        logits = jnp.dot(h.astype(jnp.bfloat16), w2_ref[...],
                         preferred_element_type=jnp.float32) + b2_ref[...]
        o_ref[...] = logits.astype(o_ref.dtype)


def _mlp_head(x, w1, b1, w2, b2, *, k_chunk=2048, b_block=256):
    B, K = x.shape
    b_block = min(b_block, B)
    n_hidden = w1.shape[1]
    n_out = w2.shape[1]
    return pl.pallas_call(
        _mlp_kernel,
        out_shape=jax.ShapeDtypeStruct((B, n_out), jnp.float32),
        grid_spec=pltpu.PrefetchScalarGridSpec(
            num_scalar_prefetch=0,
            grid=(B // b_block, K // k_chunk),
            in_specs=[
                pl.BlockSpec((b_block, k_chunk), lambda i, k: (i, k)),
                pl.BlockSpec((k_chunk, n_hidden), lambda i, k: (k, 0)),
                pl.BlockSpec((1, n_hidden), lambda i, k: (0, 0)),
                pl.BlockSpec((n_hidden, n_out), lambda i, k: (0, 0)),
                pl.BlockSpec((1, n_out), lambda i, k: (0, 0)),
            ],
            out_specs=pl.BlockSpec((b_block, n_out), lambda i, k: (i, 0)),
            scratch_shapes=[pltpu.VMEM((b_block, n_hidden), jnp.float32)],
        ),
        compiler_params=pltpu.CompilerParams(
            dimension_semantics=("parallel", "arbitrary"),
        ),
    )(x, w1, b1, w2, b2)


def kernel(w1, b1, w2, b2, fc1_w, fc1_b, fc2_w, fc2_b, x_nchw):
    t1 = _conv1_tables(w1)
    t2 = _conv2_tables(w2)
    b1t = jnp.tile(b1, (1, 32))
    b2t = jnp.tile(b2, (1, 4))
    h = _conv_stage(x_nchw, t1, b1t, t2, b2t, _row_select_1(),
                    _row_select_2())
    flat = h.reshape(h.shape[0], 16 * 16 * 64)
    return _mlp_head(flat, fc1_w.astype(jnp.bfloat16), fc1_b,
                     fc2_w.astype(jnp.bfloat16), fc2_b)
